# Initial kernel scaffold; baseline (speedup 1.0000x reference)
import jax
import jax.numpy as jnp
from jax.experimental import pallas as pl


def kernel(x, edge_index, batch, W0, b0, W1, b1, W2, b2, Wfc, bfc):
    raise NotImplementedError("write your pallas kernel here")


# speedup vs baseline: 8.8991x; 8.8991x over previous
import functools
import jax
import jax.numpy as jnp
from jax import lax
from jax.experimental import pallas as pl
from jax.experimental.pallas import tpu as pltpu
from jax.experimental.pallas import tpu_sc as plsc

NN = 10000
EE = 320000
DD = 128
HH = 64
GG = 100
CC = 10

NCORE = 2
NSUB = 16
NWORK = NCORE * NSUB
EPW = EE // NWORK
CHUNK = 80
NCHUNK = EPW // CHUNK
RB = 1000
NRB = NN // RB
NPAD = 10240
RPS = NPAD // NSUB
DW = 16


def _sigmoid(t):
    return 1.0 / (1.0 + jnp.exp(-t))


def _make_sc_segsum(width, with_deg):
    out_type = [jax.ShapeDtypeStruct((NCORE, NPAD, width), jnp.float32)]
    scratch = [
        pltpu.VMEM((NCHUNK, CHUNK), jnp.int32),
        pltpu.VMEM((NCHUNK, CHUNK), jnp.int32),
        pltpu.VMEM((CHUNK, width), jnp.float32),
        pltpu.VMEM_SHARED((NPAD, width), jnp.float32),
        pltpu.SemaphoreType.DMA,
    ]
    if with_deg:
        out_type.append(jax.ShapeDtypeStruct((NCORE, NPAD, DW), jnp.float32))
        scratch += [
            pltpu.VMEM((CHUNK, DW), jnp.float32),
            pltpu.VMEM_SHARED((NPAD, DW), jnp.float32),
        ]

    @functools.partial(
        pl.kernel,
        out_type=out_type,
        mesh=plsc.VectorSubcoreMesh(core_axis_name="c", subcore_axis_name="s"),
        compiler_params=pltpu.CompilerParams(use_tc_tiling_on_sc=False),
        scratch_types=scratch,
    )
    def sc_segsum(val_hbm, srcr_hbm, dstr_hbm, zeros_hbm, *rest):
        if with_deg:
            (zerosd_hbm, ones_hbm, out_hbm, outd_hbm,
             src_v, dst_v, rows_v, acc, sem, ones_v, accd) = rest
        else:
            out_hbm, src_v, dst_v, rows_v, acc, sem = rest
        c = lax.axis_index("c")
        s = lax.axis_index("s")
        wid = c * NSUB + s
        pltpu.sync_copy(zeros_hbm.at[pl.ds(s * RPS, RPS)],
                        acc.at[pl.ds(s * RPS, RPS)])
        pltpu.sync_copy(srcr_hbm.at[wid], src_v)
        pltpu.sync_copy(dstr_hbm.at[wid], dst_v)
        if with_deg:
            pltpu.sync_copy(zerosd_hbm.at[pl.ds(s * RPS, RPS)],
                            accd.at[pl.ds(s * RPS, RPS)])
            pltpu.sync_copy(ones_hbm, ones_v)
        plsc.subcore_barrier()

        def body(i, carry):
            pltpu.async_copy(val_hbm.at[src_v.at[i]], rows_v, sem).wait()
            pltpu.sync_copy(rows_v, acc.at[dst_v.at[i]], add=True)
            if with_deg:
                pltpu.sync_copy(ones_v, accd.at[dst_v.at[i]], add=True)
            return carry

        lax.fori_loop(0, NCHUNK, body, 0)
        plsc.subcore_barrier()
        pltpu.sync_copy(acc.at[pl.ds(s * RPS, RPS)],
                        out_hbm.at[c, pl.ds(s * RPS, RPS)])
        if with_deg:
            pltpu.sync_copy(accd.at[pl.ds(s * RPS, RPS)],
                            outd_hbm.at[c, pl.ds(s * RPS, RPS)])

    return sc_segsum


_sc_segsum_l0 = _make_sc_segsum(DD, True)
_sc_segsum_h = _make_sc_segsum(HH, False)


def _make_tc_layer(width):
    def body(h_ref, p0_ref, p1_ref, d0_ref, d1_ref, b_ref, w_ref, out_ref):
        h = h_ref[...]
        aggh = p0_ref[0] + p1_ref[0]
        deg = d0_ref[0][:, 0:1] + d1_ref[0][:, 0:1]
        u = 2.0 * h + aggh
        v = 2.0 * (1.0 - h) + (deg - aggh)
        xh = jnp.concatenate([u, v], axis=1)
        t = jnp.dot(xh, w_ref[...], preferred_element_type=jnp.float32) + b_ref[...]
        out_ref[...] = _sigmoid(t)

    def call(h, pa, dg, b_row, W):
        return pl.pallas_call(
            body,
            grid=(NRB,),
            in_specs=[
                pl.BlockSpec((RB, width), lambda i: (i, 0)),
                pl.BlockSpec((1, RB, width), lambda i: (0, i, 0)),
                pl.BlockSpec((1, RB, width), lambda i: (1, i, 0)),
                pl.BlockSpec((1, RB, DW), lambda i: (0, i, 0)),
                pl.BlockSpec((1, RB, DW), lambda i: (1, i, 0)),
                pl.BlockSpec((1, HH), lambda i: (0, 0)),
                pl.BlockSpec((2 * width, HH), lambda i: (0, 0)),
            ],
            out_specs=pl.BlockSpec((RB, HH), lambda i: (i, 0)),
            out_shape=jax.ShapeDtypeStruct((NN, HH), jnp.float32),
        )(h, pa, pa, dg, dg, b_row, W)

    return call


_tc_layer0 = _make_tc_layer(DD)
_tc_layerh = _make_tc_layer(HH)


def _pool_body(h1_ref, h2_ref, h3_ref, bat_ref, bcol_ref,
               ssum_ref, smax_ref, cnt_ref):
    i = pl.program_id(0)

    @pl.when(i == 0)
    def _():
        ssum_ref[...] = jnp.zeros_like(ssum_ref)
        smax_ref[...] = jnp.zeros_like(smax_ref)
        cnt_ref[...] = jnp.zeros_like(cnt_ref)

    xb = jnp.concatenate([h1_ref[...], h2_ref[...], h3_ref[...]], axis=1)
    bb = bat_ref[0]
    bcol = bcol_ref[...]

    xs = xb
    s = 1
    while s < RB:
        bshift = jnp.concatenate(
            [jnp.full((s, 1), -1, jnp.int32), bcol[:RB - s]], axis=0)
        ok = bcol == bshift
        xshift = jnp.concatenate(
            [jnp.zeros((s, 3 * HH), jnp.float32), xs[:RB - s]], axis=0)
        xs = jnp.maximum(xs, jnp.where(ok, xshift, 0.0))
        s *= 2

    gid = lax.broadcasted_iota(jnp.int32, (GG, RB), 0)
    m = (bb == gid).astype(jnp.float32)
    islast = jnp.concatenate(
        [(bb[:, 1:] != bb[:, :-1]).astype(jnp.float32),
         jnp.ones((1, 1), jnp.float32)], axis=1)
    msel = m * islast

    cnt_ref[...] += jnp.sum(m, axis=1, keepdims=True) * jnp.ones(
        (1, 128), jnp.float32)
    ssum_ref[...] += jnp.dot(m, xb, preferred_element_type=jnp.float32)
    smax_ref[...] = jnp.maximum(
        smax_ref[...], jnp.dot(msel, xs, preferred_element_type=jnp.float32))


def _tc_pool(h1, h2, h3, bat3, bcol):
    return pl.pallas_call(
        _pool_body,
        grid=(NRB,),
        in_specs=[
            pl.BlockSpec((RB, HH), lambda i: (i, 0)),
            pl.BlockSpec((RB, HH), lambda i: (i, 0)),
            pl.BlockSpec((RB, HH), lambda i: (i, 0)),
            pl.BlockSpec((1, 1, RB), lambda i: (i, 0, 0)),
            pl.BlockSpec((RB, 1), lambda i: (i, 0)),
        ],
        out_specs=[
            pl.BlockSpec((GG, 3 * HH), lambda i: (0, 0)),
            pl.BlockSpec((GG, 3 * HH), lambda i: (0, 0)),
            pl.BlockSpec((GG, 128), lambda i: (0, 0)),
        ],
        out_shape=[
            jax.ShapeDtypeStruct((GG, 3 * HH), jnp.float32),
            jax.ShapeDtypeStruct((GG, 3 * HH), jnp.float32),
            jax.ShapeDtypeStruct((GG, 128), jnp.float32),
        ],
    )(h1, h2, h3, bat3, bcol)


def _head_body(ssum_ref, smax_ref, cnt_ref, w_ref, b_ref, out_ref):
    ssum = ssum_ref[...]
    cnt = cnt_ref[:, 0:1]
    mean = ssum / jnp.maximum(cnt, 1.0)
    g = jnp.concatenate([mean, smax_ref[...], ssum], axis=1)
    gf = jnp.concatenate([g, 1.0 - g], axis=1)
    t = jnp.dot(gf, w_ref[...], preferred_element_type=jnp.float32) + b_ref[...]
    out_ref[...] = _sigmoid(t)


def _tc_head(ssum, smax, cnt, Wfc, bfc_row):
    return pl.pallas_call(
        _head_body,
        in_specs=[
            pl.BlockSpec((GG, 3 * HH), lambda: (0, 0)),
            pl.BlockSpec((GG, 3 * HH), lambda: (0, 0)),
            pl.BlockSpec((GG, 128), lambda: (0, 0)),
            pl.BlockSpec((18 * HH, CC), lambda: (0, 0)),
            pl.BlockSpec((1, CC), lambda: (0, 0)),
        ],
        out_specs=pl.BlockSpec((GG, CC), lambda: (0, 0)),
        out_shape=jax.ShapeDtypeStruct((GG, CC), jnp.float32),
    )(ssum, smax, cnt, Wfc, bfc_row)


def kernel(x, edge_index, batch, W0, b0, W1, b1, W2, b2, Wfc, bfc):
    src = edge_index[0].reshape(NWORK, NCHUNK, CHUNK)
    dst = edge_index[1].reshape(NWORK, NCHUNK, CHUNK)
    zeros0 = jnp.zeros((NPAD, DD), jnp.float32)
    zerosh = jnp.zeros((NPAD, HH), jnp.float32)
    zerosd = jnp.zeros((NPAD, DW), jnp.float32)
    ones = jnp.ones((CHUNK, DW), jnp.float32)
    bat3 = batch.reshape(NRB, 1, RB)
    bcol = batch.reshape(NN, 1)
    b0r = b0.reshape(1, HH)
    b1r = b1.reshape(1, HH)
    b2r = b2.reshape(1, HH)
    bfcr = bfc.reshape(1, CC)

    pa, dg = _sc_segsum_l0(x, src, dst, zeros0, zerosd, ones)
    h1 = _tc_layer0(x, pa, dg, b0r, W0)
    (pa2,) = _sc_segsum_h(h1, src, dst, zerosh)
    h2 = _tc_layerh(h1, pa2, dg, b1r, W1)
    (pa3,) = _sc_segsum_h(h2, src, dst, zerosh)
    h3 = _tc_layerh(h2, pa3, dg, b2r, W2)
    ssum, smax, cnt = _tc_pool(h1, h2, h3, bat3, bcol)
    return _tc_head(ssum, smax, cnt, Wfc, bfcr)


# speedup vs baseline: 14.9060x; 1.6750x over previous
import functools
import jax
import jax.numpy as jnp
from jax import lax
from jax.experimental import pallas as pl
from jax.experimental.pallas import tpu as pltpu
from jax.experimental.pallas import tpu_sc as plsc

NN = 10000
EE = 320000
DD = 128
HH = 64
GG = 100
CC = 10

NCORE = 2
NSUB = 16
NWORK = NCORE * NSUB
EPW = EE // NWORK
CHUNK = 80
NCHUNK = EPW // CHUNK
RB = 1000
NRB = NN // RB
NPAD = 10240
RPS = NPAD // NSUB


def _sigmoid(t):
    return 1.0 / (1.0 + jnp.exp(-t))


def _first_body(x_ref, w_ref, z_ref):
    xb = x_ref[...]
    xh = jnp.concatenate([xb, 1.0 - xb], axis=1)
    z_ref[...] = jnp.dot(xh, w_ref[...], preferred_element_type=jnp.float32,
                         precision=lax.Precision.HIGHEST)


def _tc_first(x, W0):
    return pl.pallas_call(
        _first_body,
        grid=(NRB,),
        in_specs=[
            pl.BlockSpec((RB, DD), lambda i: (i, 0)),
            pl.BlockSpec((2 * DD, HH), lambda i: (0, 0)),
        ],
        out_specs=pl.BlockSpec((RB, HH), lambda i: (i, 0)),
        out_shape=jax.ShapeDtypeStruct((NN, HH), jnp.float32),
    )(x, W0)


def _mid_body(z_ref, p0_ref, p1_ref, b_ref, w_ref, h_ref, zn_ref):
    t = 2.0 * z_ref[...] + p0_ref[0] + p1_ref[0] + b_ref[...]
    h = _sigmoid(t)
    h_ref[...] = h
    xh = jnp.concatenate([h, 1.0 - h], axis=1)
    zn_ref[...] = jnp.dot(xh, w_ref[...], preferred_element_type=jnp.float32,
                          precision=lax.Precision.HIGHEST)


def _tc_mid(z, pa, b_row, Wn):
    return pl.pallas_call(
        _mid_body,
        grid=(NRB,),
        in_specs=[
            pl.BlockSpec((RB, HH), lambda i: (i, 0)),
            pl.BlockSpec((1, RB, HH), lambda i: (0, i, 0)),
            pl.BlockSpec((1, RB, HH), lambda i: (1, i, 0)),
            pl.BlockSpec((1, HH), lambda i: (0, 0)),
            pl.BlockSpec((2 * HH, HH), lambda i: (0, 0)),
        ],
        out_specs=[
            pl.BlockSpec((RB, HH), lambda i: (i, 0)),
            pl.BlockSpec((RB, HH), lambda i: (i, 0)),
        ],
        out_shape=[
            jax.ShapeDtypeStruct((NN, HH), jnp.float32),
            jax.ShapeDtypeStruct((NN, HH), jnp.float32),
        ],
    )(z, pa, pa, b_row, Wn)


def _pool_body(h1_ref, h2_ref, z2_ref, p0_ref, p1_ref, b_ref, bat_ref,
               bcol_ref, ssum_ref, smax_ref, cnt_ref):
    i = pl.program_id(0)

    @pl.when(i == 0)
    def _():
        ssum_ref[...] = jnp.zeros_like(ssum_ref)
        smax_ref[...] = jnp.zeros_like(smax_ref)
        cnt_ref[...] = jnp.zeros_like(cnt_ref)

    t = 2.0 * z2_ref[...] + p0_ref[0] + p1_ref[0] + b_ref[...]
    h3 = _sigmoid(t)
    xb = jnp.concatenate([h1_ref[...], h2_ref[...], h3], axis=1)
    bb = bat_ref[0]
    bcol = bcol_ref[...]

    xs = xb
    s = 1
    while s < RB:
        bshift = jnp.concatenate(
            [jnp.full((s, 1), -1, jnp.int32), bcol[:RB - s]], axis=0)
        ok = bcol == bshift
        xshift = jnp.concatenate(
            [jnp.zeros((s, 3 * HH), jnp.float32), xs[:RB - s]], axis=0)
        xs = jnp.maximum(xs, jnp.where(ok, xshift, 0.0))
        s *= 2

    gid = lax.broadcasted_iota(jnp.int32, (GG, RB), 0)
    m = (bb == gid).astype(jnp.float32)
    islast = jnp.concatenate(
        [(bb[:, 1:] != bb[:, :-1]).astype(jnp.float32),
         jnp.ones((1, 1), jnp.float32)], axis=1)
    msel = m * islast

    cnt_ref[...] += jnp.sum(m, axis=1, keepdims=True) * jnp.ones(
        (1, 128), jnp.float32)
    ssum_ref[...] += jnp.dot(m, xb, preferred_element_type=jnp.float32)
    smax_ref[...] = jnp.maximum(
        smax_ref[...], jnp.dot(msel, xs, preferred_element_type=jnp.float32))


def _tc_pool(h1, h2, z2, pa, b_row, bat3, bcol):
    return pl.pallas_call(
        _pool_body,
        grid=(NRB,),
        in_specs=[
            pl.BlockSpec((RB, HH), lambda i: (i, 0)),
            pl.BlockSpec((RB, HH), lambda i: (i, 0)),
            pl.BlockSpec((RB, HH), lambda i: (i, 0)),
            pl.BlockSpec((1, RB, HH), lambda i: (0, i, 0)),
            pl.BlockSpec((1, RB, HH), lambda i: (1, i, 0)),
            pl.BlockSpec((1, HH), lambda i: (0, 0)),
            pl.BlockSpec((1, 1, RB), lambda i: (i, 0, 0)),
            pl.BlockSpec((RB, 1), lambda i: (i, 0)),
        ],
        out_specs=[
            pl.BlockSpec((GG, 3 * HH), lambda i: (0, 0)),
            pl.BlockSpec((GG, 3 * HH), lambda i: (0, 0)),
            pl.BlockSpec((GG, 128), lambda i: (0, 0)),
        ],
        out_shape=[
            jax.ShapeDtypeStruct((GG, 3 * HH), jnp.float32),
            jax.ShapeDtypeStruct((GG, 3 * HH), jnp.float32),
            jax.ShapeDtypeStruct((GG, 128), jnp.float32),
        ],
    )(h1, h2, z2, pa, pa, b_row, bat3, bcol)


def _head_body(ssum_ref, smax_ref, cnt_ref, w_ref, b_ref, out_ref):
    ssum = ssum_ref[...]
    cnt = cnt_ref[:, 0:1]
    mean = ssum / jnp.maximum(cnt, 1.0)
    g = jnp.concatenate([mean, smax_ref[...], ssum], axis=1)
    gf = jnp.concatenate([g, 1.0 - g], axis=1)
    t = jnp.dot(gf, w_ref[...], preferred_element_type=jnp.float32) + b_ref[...]
    out_ref[...] = _sigmoid(t)


def _tc_head(ssum, smax, cnt, Wfc, bfc_row):
    return pl.pallas_call(
        _head_body,
        in_specs=[
            pl.BlockSpec((GG, 3 * HH), lambda: (0, 0)),
            pl.BlockSpec((GG, 3 * HH), lambda: (0, 0)),
            pl.BlockSpec((GG, 128), lambda: (0, 0)),
            pl.BlockSpec((18 * HH, CC), lambda: (0, 0)),
            pl.BlockSpec((1, CC), lambda: (0, 0)),
        ],
        out_specs=pl.BlockSpec((GG, CC), lambda: (0, 0)),
        out_shape=jax.ShapeDtypeStruct((GG, CC), jnp.float32),
    )(ssum, smax, cnt, Wfc, bfc_row)


@functools.partial(
    pl.kernel,
    out_type=jax.ShapeDtypeStruct((NCORE, NPAD, HH), jnp.float32),
    mesh=plsc.VectorSubcoreMesh(core_axis_name="c", subcore_axis_name="s"),
    compiler_params=pltpu.CompilerParams(use_tc_tiling_on_sc=False),
    scratch_types=[
        pltpu.VMEM((NCHUNK, CHUNK), jnp.int32),
        pltpu.VMEM((NCHUNK, CHUNK), jnp.int32),
        pltpu.VMEM((CHUNK, HH), jnp.float32),
        pltpu.VMEM((CHUNK, HH), jnp.float32),
        pltpu.VMEM_SHARED((NPAD, HH), jnp.float32),
        pltpu.SemaphoreType.DMA,
        pltpu.SemaphoreType.DMA,
    ],
)
def _sc_segsum(z_hbm, srcr_hbm, dstr_hbm, zeros_hbm, out_hbm,
               src_v, dst_v, rows0, rows1, acc, sem0, sem1):
    c = lax.axis_index("c")
    s = lax.axis_index("s")
    wid = c * NSUB + s
    pltpu.sync_copy(zeros_hbm.at[pl.ds(s * RPS, RPS)],
                    acc.at[pl.ds(s * RPS, RPS)])
    pltpu.sync_copy(srcr_hbm.at[wid], src_v)
    pltpu.sync_copy(dstr_hbm.at[wid], dst_v)
    plsc.subcore_barrier()

    HALF = NCHUNK // 2
    pltpu.async_copy(z_hbm.at[src_v.at[0]], rows0, sem0)

    def body(g, carry):
        i0 = 2 * g
        pltpu.async_copy(z_hbm.at[src_v.at[i0 + 1]], rows1, sem1)
        pltpu.make_async_copy(z_hbm.at[src_v.at[i0]], rows0, sem0).wait()
        pltpu.sync_copy(rows0, acc.at[dst_v.at[i0]], add=True)
        pltpu.async_copy(z_hbm.at[src_v.at[i0 + 2]], rows0, sem0)
        pltpu.make_async_copy(z_hbm.at[src_v.at[i0 + 1]], rows1, sem1).wait()
        pltpu.sync_copy(rows1, acc.at[dst_v.at[i0 + 1]], add=True)
        return carry

    lax.fori_loop(0, HALF, body, 0)
    pltpu.make_async_copy(z_hbm.at[src_v.at[NCHUNK - 1]], rows0, sem0).wait()
    pltpu.sync_copy(rows0, acc.at[dst_v.at[NCHUNK - 1]], add=True)

    plsc.subcore_barrier()
    pltpu.sync_copy(acc.at[pl.ds(s * RPS, RPS)],
                    out_hbm.at[c, pl.ds(s * RPS, RPS)])


def kernel(x, edge_index, batch, W0, b0, W1, b1, W2, b2, Wfc, bfc):
    src = edge_index[0].reshape(NWORK, NCHUNK, CHUNK)
    dst = edge_index[1].reshape(NWORK, NCHUNK, CHUNK)
    zeros = jnp.zeros((NPAD, HH), jnp.float32)
    bat3 = batch.reshape(NRB, 1, RB)
    bcol = batch.reshape(NN, 1)
    b0r = b0.reshape(1, HH)
    b1r = b1.reshape(1, HH)
    b2r = b2.reshape(1, HH)
    bfcr = bfc.reshape(1, CC)

    z0 = _tc_first(x, W0)
    pa = _sc_segsum(z0, src, dst, zeros)
    h1, z1 = _tc_mid(z0, pa, b0r, W1)
    pa = _sc_segsum(z1, src, dst, zeros)
    h2, z2 = _tc_mid(z1, pa, b1r, W2)
    pa = _sc_segsum(z2, src, dst, zeros)
    ssum, smax, cnt = _tc_pool(h1, h2, z2, pa, b2r, bat3, bcol)
    return _tc_head(ssum, smax, cnt, Wfc, bfcr)
